# baseline (speedup 1.0000x reference)
import jax
import jax.numpy as jnp
from jax.experimental import pallas as pl
from jax.experimental.pallas import tpu as pltpu

F32 = jnp.float32
BF16 = jnp.bfloat16


def _ln_proj_body(x_ref, lnw_ref, lnb_ref, wu_ref, wz_ref, u_ref, z_ref):
    x = x_ref[...]
    mu = jnp.mean(x, axis=-1, keepdims=True)
    xc = x - mu
    var = jnp.mean(xc * xc, axis=-1, keepdims=True)
    xn = xc * jax.lax.rsqrt(var + 1e-5) * lnw_ref[...] + lnb_ref[...]
    xb = xn.astype(BF16)
    u_ref[...] = jnp.dot(xb, wu_ref[...], preferred_element_type=F32).astype(BF16)
    z_ref[...] = jnp.dot(xb, wz_ref[...], preferred_element_type=F32).astype(BF16)


def _ln_proj(x2, ln_w, ln_b, w_u, w_z, mb):
    bl, d = x2.shape
    di = w_u.shape[1]
    grid = (bl // mb,)
    return pl.pallas_call(
        _ln_proj_body,
        grid=grid,
        in_specs=[
            pl.BlockSpec((mb, d), lambda i: (i, 0)),
            pl.BlockSpec((1, d), lambda i: (0, 0)),
            pl.BlockSpec((1, d), lambda i: (0, 0)),
            pl.BlockSpec((d, di), lambda i: (0, 0)),
            pl.BlockSpec((d, di), lambda i: (0, 0)),
        ],
        out_specs=[
            pl.BlockSpec((mb, di), lambda i: (i, 0)),
            pl.BlockSpec((mb, di), lambda i: (i, 0)),
        ],
        out_shape=[
            jax.ShapeDtypeStruct((bl, di), BF16),
            jax.ShapeDtypeStruct((bl, di), BF16),
        ],
        compiler_params=pltpu.CompilerParams(
            dimension_semantics=("parallel",),
            vmem_limit_bytes=56 * 1024 * 1024,
        ),
        name="mamba_ln_proj",
    )(x2, ln_w, ln_b, w_u, w_z)


def _scan_body(u_ref, z_ref, sh_ref, cw_ref, cb_ref, wx_ref, wdt_ref, bdt_ref,
               at_ref, dp_ref, y_ref,
               prev_ref, h_ref, ue_s, dt_s, u_s, hs_s, bm_s):
    j = pl.program_id(1)
    lc, di = u_ref.shape[1], u_ref.shape[2]
    k = cw_ref.shape[0]
    ds = at_ref.shape[0]
    dtr = wdt_ref.shape[0]

    @pl.when(j == 0)
    def _():
        prev_ref[...] = jnp.zeros_like(prev_ref)
        h_ref[...] = jnp.zeros_like(h_ref)

    urb = u_ref[0]
    ur = urb.astype(F32)

    ue_s[0:8] = prev_ref[...]
    ue_s[8:] = urb
    prev_ref[...] = urb[lc - 8:]
    stack = jnp.dot(sh_ref[...], ue_s[...], preferred_element_type=F32)
    uc = cw_ref[k - 1:k, :] * ur
    for q in range(k - 1):
        uc = uc + cw_ref[q:q + 1, :] * stack[q * lc:(q + 1) * lc]
    u = jax.nn.silu(uc + cb_ref[...])

    ub = u.astype(BF16)
    xd = jnp.dot(ub, wx_ref[...], preferred_element_type=F32)
    dt_raw = jnp.dot(xd[:, :dtr].astype(BF16), wdt_ref[...],
                     preferred_element_type=F32) + bdt_ref[...]
    dt = jax.nn.softplus(dt_raw)
    dt_s[...] = dt
    u_s[...] = u
    bm_s[...] = xd[:, dtr:dtr + ds]
    cm = xd[:, dtr + ds:dtr + 2 * ds]
    at = at_ref[...]

    jj = jax.lax.broadcasted_iota(jnp.int32, (lc, lc * ds), 1)
    tt = jax.lax.broadcasted_iota(jnp.int32, (lc, lc * ds), 0)
    cmrep = pltpu.repeat(cm, lc, axis=1)
    g = jnp.where((jj // ds) == tt, cmrep, 0.0).astype(BF16)

    def sub_chunk(c, h):
        dt8 = dt_s[pl.ds(c * 8, 8)]
        u8 = u_s[pl.ds(c * 8, 8)]
        bm8t = bm_s[pl.ds(c * 8, 8), :].T
        for t in range(8):
            dt_t = dt8[t:t + 1]
            da = jnp.exp(dt_t * at)
            dbu = (dt_t * u8[t:t + 1]) * bm8t[:, t:t + 1]
            h = h * da + dbu
            base = pl.multiple_of((c * 8 + t) * ds, ds)
            hs_s[pl.ds(base, ds)] = h.astype(BF16)
        return h

    h = jax.lax.fori_loop(0, lc // 8, sub_chunk, h_ref[...])
    h_ref[...] = h

    ys = jnp.dot(g, hs_s[...], preferred_element_type=F32)
    z = z_ref[0].astype(F32)
    y = (ys + u_s[...] * dp_ref[...]) * jax.nn.silu(z)
    y_ref[0] = y.astype(BF16)


def _scan(u_raw, z, sh, conv_w_t, conv_b, w_x, w_dt, b_dt, a_t, dp, lc):
    b, l, di = u_raw.shape
    k = conv_w_t.shape[0]
    ds = a_t.shape[0]
    dtr = w_dt.shape[0]
    nx = w_x.shape[1]
    grid = (b, l // lc)
    blk3 = pl.BlockSpec((1, lc, di), lambda i, j: (i, j, 0))
    full = lambda shape: pl.BlockSpec(shape, lambda i, j: tuple(0 for _ in shape))
    return pl.pallas_call(
        _scan_body,
        grid=grid,
        in_specs=[
            blk3, blk3,
            full(sh.shape),
            full((k, di)), full((1, di)), full((di, nx)), full((dtr, di)),
            full((1, di)), full((ds, di)), full((1, di)),
        ],
        out_specs=blk3,
        out_shape=jax.ShapeDtypeStruct((b, l, di), BF16),
        scratch_shapes=[
            pltpu.VMEM((8, di), BF16),
            pltpu.VMEM((ds, di), F32),
            pltpu.VMEM((8 + lc, di), BF16),
            pltpu.VMEM((lc, di), F32),
            pltpu.VMEM((lc, di), F32),
            pltpu.VMEM((lc * ds, di), BF16),
            pltpu.VMEM((lc, ds), F32),
        ],
        compiler_params=pltpu.CompilerParams(
            dimension_semantics=("parallel", "arbitrary"),
            vmem_limit_bytes=56 * 1024 * 1024,
        ),
        name="mamba_scan",
    )(u_raw, z, sh, conv_w_t, conv_b, w_x, w_dt, b_dt, a_t, dp)


def _out_proj_body(y_ref, wo_ref, x_ref, o_ref):
    o_ref[...] = jnp.dot(y_ref[...], wo_ref[...],
                         preferred_element_type=F32) + x_ref[...]


def _out_proj(y2, w_o, x2, mb):
    bl, di = y2.shape
    d = w_o.shape[1]
    grid = (bl // mb,)
    return pl.pallas_call(
        _out_proj_body,
        grid=grid,
        in_specs=[
            pl.BlockSpec((mb, di), lambda i: (i, 0)),
            pl.BlockSpec((di, d), lambda i: (0, 0)),
            pl.BlockSpec((mb, d), lambda i: (i, 0)),
        ],
        out_specs=pl.BlockSpec((mb, d), lambda i: (i, 0)),
        out_shape=jax.ShapeDtypeStruct((bl, d), F32),
        compiler_params=pltpu.CompilerParams(
            dimension_semantics=("parallel",),
            vmem_limit_bytes=56 * 1024 * 1024,
        ),
        name="mamba_out_proj",
    )(y2, w_o, x2)


def kernel(x, ln_w, ln_b, W_in, conv_w, conv_b, W_x, W_dt, b_dt, A_log, Dp, W_out):
    b, l, d = x.shape
    di = W_in.shape[1] // 2

    mb = min(1024, b * l)
    lc = min(128, l)
    x2 = x.reshape(b * l, d)
    w_u = W_in[:, :di].astype(BF16)
    w_z = W_in[:, di:].astype(BF16)
    u_raw, z = _ln_proj(x2, ln_w.reshape(1, d), ln_b.reshape(1, d),
                        w_u, w_z, mb=mb)

    conv_w_t = conv_w[:, 0, :].T
    a_t = (-jnp.exp(A_log)).T
    ksz = conv_w.shape[2]
    rows = jnp.arange((ksz - 1) * lc)
    qq, ii = rows // lc, rows % lc
    cols = 8 + ii - (ksz - 1 - qq)
    sh = (jnp.arange(8 + lc)[None, :] == cols[:, None]).astype(BF16)
    y = _scan(u_raw.reshape(b, l, di), z.reshape(b, l, di),
              sh, conv_w_t, conv_b.reshape(1, di),
              W_x.astype(BF16), W_dt.astype(BF16), b_dt.reshape(1, di),
              a_t, Dp.reshape(1, di), lc=lc)

    out = _out_proj(y.reshape(b * l, di), W_out.astype(BF16), x2, mb=mb)
    return out.reshape(b, l, d)


# speedup vs baseline: 1.0589x; 1.0589x over previous
import jax
import jax.numpy as jnp
from jax.experimental import pallas as pl
from jax.experimental.pallas import tpu as pltpu

F32 = jnp.float32
BF16 = jnp.bfloat16


def _ln_proj_body(x_ref, lnw_ref, lnb_ref, w_ref, xz_ref):
    x = x_ref[...]
    mu = jnp.mean(x, axis=-1, keepdims=True)
    xc = x - mu
    var = jnp.mean(xc * xc, axis=-1, keepdims=True)
    xn = xc * jax.lax.rsqrt(var + 1e-5) * lnw_ref[...] + lnb_ref[...]
    xz_ref[...] = jnp.dot(xn.astype(BF16), w_ref[...],
                          preferred_element_type=F32).astype(BF16)


def _ln_proj(x2, ln_w, ln_b, w_in, mb):
    bl, d = x2.shape
    n = w_in.shape[1]
    grid = (bl // mb,)
    return pl.pallas_call(
        _ln_proj_body,
        grid=grid,
        in_specs=[
            pl.BlockSpec((mb, d), lambda i: (i, 0)),
            pl.BlockSpec((1, d), lambda i: (0, 0)),
            pl.BlockSpec((1, d), lambda i: (0, 0)),
            pl.BlockSpec((d, n), lambda i: (0, 0)),
        ],
        out_specs=pl.BlockSpec((mb, n), lambda i: (i, 0)),
        out_shape=jax.ShapeDtypeStruct((bl, n), BF16),
        compiler_params=pltpu.CompilerParams(
            dimension_semantics=("parallel",),
            vmem_limit_bytes=56 * 1024 * 1024,
        ),
        name="mamba_ln_proj",
    )(x2, ln_w, ln_b, w_in)


def _scan_body(u_ref, z_ref, sh_ref, cw_ref, cb_ref, wx_ref, wdt_ref, bdt_ref,
               at_ref, dp_ref, y_ref,
               prev_ref, h_ref, ue_s, dt_s, u_s, dtu_s, hs_s, bm_s):
    j = pl.program_id(1)
    lc, di = u_ref.shape[1], u_ref.shape[2]
    k = cw_ref.shape[0]
    ds = at_ref.shape[0]
    dtr = wdt_ref.shape[0]

    @pl.when(j == 0)
    def _():
        prev_ref[...] = jnp.zeros_like(prev_ref)
        h_ref[...] = jnp.zeros_like(h_ref)

    urb = u_ref[0]
    ur = urb.astype(F32)

    ue_s[0:8] = prev_ref[...]
    ue_s[8:] = urb
    prev_ref[...] = urb[lc - 8:]
    stack = jnp.dot(sh_ref[...], ue_s[...], preferred_element_type=F32)
    uc = cw_ref[k - 1:k, :] * ur
    for q in range(k - 1):
        uc = uc + cw_ref[q:q + 1, :] * stack[q * lc:(q + 1) * lc]
    u = jax.nn.silu(uc + cb_ref[...])

    ub = u.astype(BF16)
    xd = jnp.dot(ub, wx_ref[...], preferred_element_type=F32)
    dt_raw = jnp.dot(xd[:, :dtr].astype(BF16), wdt_ref[...],
                     preferred_element_type=F32) + bdt_ref[...]
    dt = jax.nn.softplus(dt_raw)
    dt_s[...] = dt
    u_s[...] = u
    dtu_s[...] = dt * u
    bm_s[...] = xd[:, dtr:dtr + ds]
    cm = xd[:, dtr + ds:dtr + 2 * ds]
    at = at_ref[...]

    jj = jax.lax.broadcasted_iota(jnp.int32, (lc, lc * ds), 1)
    tt = jax.lax.broadcasted_iota(jnp.int32, (lc, lc * ds), 0)
    cmrep = pltpu.repeat(cm, lc, axis=1)
    g = jnp.where((jj // ds) == tt, cmrep, 0.0).astype(BF16)

    hd = ds // 2

    def sub_chunk(c, carry):
        dt8 = dt_s[pl.ds(c * 8, 8)]
        u8 = dtu_s[pl.ds(c * 8, 8)]
        bm8t = bm_s[pl.ds(c * 8, 8), :].T
        h0 = h_ref[:hd]
        h1 = h_ref[hd:]
        for t in range(8):
            dt_t = dt8[t:t + 1]
            dtu_t = u8[t:t + 1]
            base = pl.multiple_of((c * 8 + t) * ds, ds)
            h0 = h0 * jnp.exp(dt_t * at[:hd]) + dtu_t * bm8t[:hd, t:t + 1]
            h1 = h1 * jnp.exp(dt_t * at[hd:]) + dtu_t * bm8t[hd:, t:t + 1]
            hs_s[pl.ds(base, ds)] = jnp.concatenate(
                [h0, h1], axis=0).astype(BF16)
        h_ref[:hd] = h0
        h_ref[hd:] = h1
        return carry

    jax.lax.fori_loop(0, lc // 8, sub_chunk, 0)

    ys = jnp.dot(g, hs_s[...], preferred_element_type=F32)
    z = z_ref[0].astype(F32)
    y = (ys + u_s[...] * dp_ref[...]) * jax.nn.silu(z)
    y_ref[0] = y.astype(BF16)


def _scan(xz3, sh, conv_w_t, conv_b, w_x, w_dt, b_dt, a_t, dp, lc):
    b, l, di2 = xz3.shape
    di = di2 // 2
    k = conv_w_t.shape[0]
    ds = a_t.shape[0]
    dtr = w_dt.shape[0]
    nx = w_x.shape[1]
    grid = (b, l // lc)
    blk3 = pl.BlockSpec((1, lc, di), lambda i, j: (i, j, 0))
    blk3z = pl.BlockSpec((1, lc, di), lambda i, j: (i, j, 1))
    full = lambda shape: pl.BlockSpec(shape, lambda i, j: tuple(0 for _ in shape))
    return pl.pallas_call(
        _scan_body,
        grid=grid,
        in_specs=[
            blk3, blk3z,
            full(sh.shape),
            full((k, di)), full((1, di)), full((di, nx)), full((dtr, di)),
            full((1, di)), full((ds, di)), full((1, di)),
        ],
        out_specs=blk3,
        out_shape=jax.ShapeDtypeStruct((b, l, di), BF16),
        scratch_shapes=[
            pltpu.VMEM((8, di), BF16),
            pltpu.VMEM((ds, di), F32),
            pltpu.VMEM((8 + lc, di), BF16),
            pltpu.VMEM((lc, di), F32),
            pltpu.VMEM((lc, di), F32),
            pltpu.VMEM((lc, di), F32),
            pltpu.VMEM((lc * ds, di), BF16),
            pltpu.VMEM((lc, ds), F32),
        ],
        compiler_params=pltpu.CompilerParams(
            dimension_semantics=("parallel", "arbitrary"),
            vmem_limit_bytes=56 * 1024 * 1024,
        ),
        name="mamba_scan",
    )(xz3, xz3, sh, conv_w_t, conv_b, w_x, w_dt, b_dt, a_t, dp)


def _out_proj_body(y_ref, wo_ref, x_ref, o_ref):
    o_ref[...] = jnp.dot(y_ref[...], wo_ref[...],
                         preferred_element_type=F32) + x_ref[...]


def _out_proj(y2, w_o, x2, mb):
    bl, di = y2.shape
    d = w_o.shape[1]
    grid = (bl // mb,)
    return pl.pallas_call(
        _out_proj_body,
        grid=grid,
        in_specs=[
            pl.BlockSpec((mb, di), lambda i: (i, 0)),
            pl.BlockSpec((di, d), lambda i: (0, 0)),
            pl.BlockSpec((mb, d), lambda i: (i, 0)),
        ],
        out_specs=pl.BlockSpec((mb, d), lambda i: (i, 0)),
        out_shape=jax.ShapeDtypeStruct((bl, d), F32),
        compiler_params=pltpu.CompilerParams(
            dimension_semantics=("parallel",),
            vmem_limit_bytes=56 * 1024 * 1024,
        ),
        name="mamba_out_proj",
    )(y2, w_o, x2)


def kernel(x, ln_w, ln_b, W_in, conv_w, conv_b, W_x, W_dt, b_dt, A_log, Dp, W_out):
    b, l, d = x.shape
    di = W_in.shape[1] // 2

    mb = min(1024, b * l)
    lc = min(128, l)
    x2 = x.reshape(b * l, d)
    xz = _ln_proj(x2, ln_w.reshape(1, d), ln_b.reshape(1, d),
                  W_in.astype(BF16), mb=mb)

    conv_w_t = conv_w[:, 0, :].T
    a_t = (-jnp.exp(A_log)).T
    ksz = conv_w.shape[2]
    rows = jnp.arange((ksz - 1) * lc)
    qq, ii = rows // lc, rows % lc
    cols = 8 + ii - (ksz - 1 - qq)
    sh = (jnp.arange(8 + lc)[None, :] == cols[:, None]).astype(BF16)
    y = _scan(xz.reshape(b, l, 2 * di),
              sh, conv_w_t, conv_b.reshape(1, di),
              W_x.astype(BF16), W_dt.astype(BF16), b_dt.reshape(1, di),
              a_t, Dp.reshape(1, di), lc=lc)

    out = _out_proj(y.reshape(b * l, di), W_out.astype(BF16), x2, mb=mb)
    return out.reshape(b, l, d)
